# baseline (speedup 1.0000x reference)
import jax
import jax.numpy as jnp
from jax import lax
from jax.experimental import pallas as pl
from jax.experimental.pallas import tpu as pltpu
from jax.experimental.pallas import tpu_sc as plsc

N = 10000
E = 320000
D = 128
L = 16
NC = 2
NS = 16
NW = NC * NS
EPW = E // NW
CH = 40
NCHUNK = EPW // CH
RPS = 624
RTAIL = N - NS * RPS


def _sc_aggr_kernel(x_hbm, src_hbm, dst_hbm, ea_hbm, out_hbm,
                    sidx0, sidx1, didx0, didx1, didx2, didx3,
                    mbuf0, mbuf1, mbuf2, mbuf3,
                    gbuf0, gbuf1, sbuf0, sbuf1, acc,
                    isem0, isem1, dsem0, dsem1, dsem2, dsem3,
                    easem0, easem1, easem2, easem3,
                    gsem0, gsem1, scsem0, scsem1):
    c = lax.axis_index("c")
    s = lax.axis_index("s")
    wid = c * NS + s
    ebase = wid * EPW
    didx = (didx0, didx1, didx2, didx3)
    dsem = (dsem0, dsem1, dsem2, dsem3)
    mbuf = (mbuf0, mbuf1, mbuf2, mbuf3)
    easem = (easem0, easem1, easem2, easem3)

    for d in range(4):
        pltpu.async_copy(dst_hbm.at[wid, d], didx[d], dsem[d])
        pltpu.async_copy(ea_hbm.at[pl.ds(ebase + d * CH, CH)],
                         mbuf[d], easem[d])
    pltpu.async_copy(src_hbm.at[wid, 0], sidx0, isem0)
    pltpu.async_copy(src_hbm.at[wid, 1], sidx1, isem1)
    pltpu.make_async_copy(src_hbm.at[wid, 0], sidx0, isem0).wait()
    pltpu.async_copy(x_hbm.at[sidx0], gbuf0, gsem0)

    @pl.loop(0, CH)
    def _(r):
        for j in range(D // L):
            sbuf0[r, pl.ds(j * L, L)] = jnp.zeros((L,), jnp.float32)

    row0 = s * RPS
    nfull = RPS // CH
    rem = RPS - nfull * CH
    for j in range(nfull):
        pltpu.sync_copy(sbuf0, acc.at[pl.ds(row0 + j * CH, CH)])
    pltpu.sync_copy(sbuf0.at[pl.ds(0, rem)],
                    acc.at[pl.ds(row0 + nfull * CH, rem)])

    @pl.when(s == NS - 1)
    def _():
        pltpu.sync_copy(sbuf0.at[pl.ds(0, RTAIL)],
                        acc.at[pl.ds(NS * RPS, RTAIL)])

    plsc.subcore_barrier()

    def chunk_body(g, k):
        b = k % 2
        q, q2 = k % 4, (k + 2) % 4
        sidx, sidxo = (sidx0, sidx1) if b == 0 else (sidx1, sidx0)
        isem, isemo = (isem0, isem1) if b == 0 else (isem1, isem0)
        gb, gbo = (gbuf0, gbuf1) if b == 0 else (gbuf1, gbuf0)
        sb = sbuf0 if b == 0 else sbuf1
        gsem, gsemo = (gsem0, gsem1) if b == 0 else (gsem1, gsem0)
        scsem = scsem0 if b == 0 else scsem1
        mb = mbuf[q]

        pltpu.make_async_copy(ea_hbm.at[pl.ds(ebase + g * CH, CH)],
                              mb, easem[q]).wait()
        pltpu.make_async_copy(x_hbm.at[sidx], gb, gsem).wait()

        @pl.when(g >= 2)
        def _():
            pltpu.make_async_copy(sb, acc.at[didx[q2]], scsem).wait()

            @pl.when(g + 2 < NCHUNK)
            def _():
                pltpu.async_copy(dst_hbm.at[wid, g + 2], didx[q2], dsem[q2])
                pltpu.async_copy(ea_hbm.at[pl.ds(ebase + (g + 2) * CH, CH)],
                                 mbuf[q2], easem[q2])

        @pl.when(g + 2 < NCHUNK)
        def _():
            pltpu.async_copy(src_hbm.at[wid, g + 2], sidx, isem)

        @pl.when(g + 1 < NCHUNK)
        def _():
            pltpu.make_async_copy(src_hbm.at[wid, g + 1], sidxo, isemo).wait()
            pltpu.async_copy(x_hbm.at[sidxo], gbo, gsemo)

        @plsc.parallel_loop(0, CH, 1, unroll=4)
        def _(r):
            for j in range(D // L):
                sl = (r, pl.ds(j * L, L))
                sb[sl] = jnp.maximum(mb[sl] + gb[sl], 0.0)

        pltpu.make_async_copy(dst_hbm.at[wid, g], didx[q], dsem[q]).wait()
        pltpu.async_copy(sb, acc.at[didx[q]], scsem, add=True)

    @pl.loop(0, NCHUNK - 2, step=4)
    def _(g):
        for k in range(4):
            chunk_body(g + k, k)

    chunk_body(jnp.int32(NCHUNK - 2), 0)
    chunk_body(jnp.int32(NCHUNK - 1), 1)

    pltpu.make_async_copy(sbuf0, acc.at[didx0], scsem0).wait()
    pltpu.make_async_copy(sbuf1, acc.at[didx1], scsem1).wait()

    plsc.subcore_barrier()

    pltpu.sync_copy(acc.at[pl.ds(row0, RPS)], out_hbm.at[c, pl.ds(row0, RPS)])

    @pl.when(s == NS - 1)
    def _():
        pltpu.sync_copy(acc.at[pl.ds(NS * RPS, RTAIL)],
                        out_hbm.at[c, pl.ds(NS * RPS, RTAIL)])


def _sc_aggr(x, src2, dst2, edge_attr):
    mesh = plsc.VectorSubcoreMesh(core_axis_name="c", subcore_axis_name="s")
    k = pl.kernel(
        _sc_aggr_kernel,
        out_type=jax.ShapeDtypeStruct((NC, N, D), jnp.float32),
        mesh=mesh,
        scratch_types=[
            pltpu.VMEM((CH,), jnp.int32),
            pltpu.VMEM((CH,), jnp.int32),
            pltpu.VMEM((CH,), jnp.int32),
            pltpu.VMEM((CH,), jnp.int32),
            pltpu.VMEM((CH,), jnp.int32),
            pltpu.VMEM((CH,), jnp.int32),
            pltpu.VMEM((CH, D), jnp.float32),
            pltpu.VMEM((CH, D), jnp.float32),
            pltpu.VMEM((CH, D), jnp.float32),
            pltpu.VMEM((CH, D), jnp.float32),
            pltpu.VMEM((CH, D), jnp.float32),
            pltpu.VMEM((CH, D), jnp.float32),
            pltpu.VMEM((CH, D), jnp.float32),
            pltpu.VMEM((CH, D), jnp.float32),
            pltpu.VMEM_SHARED((N, D), jnp.float32),
            pltpu.SemaphoreType.DMA,
            pltpu.SemaphoreType.DMA,
            pltpu.SemaphoreType.DMA,
            pltpu.SemaphoreType.DMA,
            pltpu.SemaphoreType.DMA,
            pltpu.SemaphoreType.DMA,
            pltpu.SemaphoreType.DMA,
            pltpu.SemaphoreType.DMA,
            pltpu.SemaphoreType.DMA,
            pltpu.SemaphoreType.DMA,
            pltpu.SemaphoreType.DMA,
            pltpu.SemaphoreType.DMA,
            pltpu.SemaphoreType.DMA,
            pltpu.SemaphoreType.DMA,
        ],
    )
    return k(x, src2, dst2, edge_attr)


def _tc_body(x_ref, p0_ref, p1_ref, w1_ref, b1_ref, w2_ref, b2_ref,
             eps_ref, g_ref, bt_ref, o_ref):
    z = x_ref[...] * (1.0 + eps_ref[0, 0]) + p0_ref[...] + p1_ref[...]
    h = jnp.dot(z, w1_ref[...], preferred_element_type=jnp.float32) + b1_ref[...]
    h = jnp.maximum(h, 0.0)
    h = jnp.dot(h, w2_ref[...], preferred_element_type=jnp.float32) + b2_ref[...]
    mean = jnp.mean(h, axis=1, keepdims=True)
    hc = h - mean
    var = jnp.mean(hc * hc, axis=1, keepdims=True)
    hn = hc * lax.rsqrt(var + 1e-5) * g_ref[...] + bt_ref[...]
    o_ref[...] = jnp.maximum(hn, 0.0)


BLK = 1000


def _tc_mlp(x, p0, p1, W1, b1, W2, b2, eps11, gamma, beta):
    grid = (N // BLK,)
    row_spec = pl.BlockSpec((BLK, D), lambda i: (i, 0))
    full_spec = pl.BlockSpec((D, D), lambda i: (0, 0))
    vec_spec = pl.BlockSpec((1, D), lambda i: (0, 0))
    return pl.pallas_call(
        _tc_body,
        grid=grid,
        in_specs=[row_spec, row_spec, row_spec,
                  full_spec, vec_spec, full_spec, vec_spec,
                  pl.BlockSpec((1, 1), lambda i: (0, 0)),
                  vec_spec, vec_spec],
        out_specs=row_spec,
        out_shape=jax.ShapeDtypeStruct((N, D), jnp.float32),
    )(x, p0, p1, W1, b1, W2, b2, eps11, gamma, beta)


def kernel(x, edge_index, edge_attr, W1, b1, W2, b2, eps, gamma, beta):
    src2 = edge_index[0].reshape(NW, NCHUNK, CH)
    dst2 = edge_index[1].reshape(NW, NCHUNK, CH)
    parts = _sc_aggr(x, src2, dst2, edge_attr)
    eps11 = jnp.reshape(eps, (1, 1)).astype(jnp.float32)
    return _tc_mlp(x, parts[0], parts[1], W1,
                   jnp.reshape(b1, (1, D)), W2, jnp.reshape(b2, (1, D)),
                   eps11, jnp.reshape(gamma, (1, D)), jnp.reshape(beta, (1, D)))


# speedup vs baseline: 1.2908x; 1.2908x over previous
import jax
import jax.numpy as jnp
from jax import lax
from jax.experimental import pallas as pl
from jax.experimental.pallas import tpu as pltpu
from jax.experimental.pallas import tpu_sc as plsc

N = 10000
E = 320000
D = 128
L = 16
NC = 2
NS = 16
NW = NC * NS
EPW = E // NW
CH = 80
NCHUNK = EPW // CH
RPS = 624
RTAIL = N - NS * RPS


def _sc_aggr_kernel(x_hbm, src_hbm, dst_hbm, ea_hbm, out_hbm,
                    sidx0, sidx1, didx0, didx1,
                    mbuf0, mbuf1, gbuf0, gbuf1, acc,
                    isem0, isem1, dsem0, dsem1,
                    easem0, easem1, gsem0, gsem1):
    c = lax.axis_index("c")
    s = lax.axis_index("s")
    wid = c * NS + s
    ebase = wid * EPW

    pltpu.async_copy(src_hbm.at[pl.ds(ebase, CH)], sidx0, isem0)
    pltpu.async_copy(src_hbm.at[pl.ds(ebase + CH, CH)], sidx1, isem1)
    pltpu.async_copy(dst_hbm.at[pl.ds(ebase, CH)], didx0, dsem0)
    pltpu.async_copy(dst_hbm.at[pl.ds(ebase + CH, CH)], didx1, dsem1)
    pltpu.async_copy(ea_hbm.at[pl.ds(ebase, CH)], mbuf0, easem0)
    pltpu.async_copy(ea_hbm.at[pl.ds(ebase + CH, CH)], mbuf1, easem1)
    pltpu.make_async_copy(src_hbm.at[pl.ds(ebase, CH)], sidx0, isem0).wait()
    pltpu.async_copy(x_hbm.at[sidx0], gbuf0, gsem0)

    @pl.loop(0, CH)
    def _(r):
        for j in range(D // L):
            gbuf1[r, pl.ds(j * L, L)] = jnp.zeros((L,), jnp.float32)

    row0 = s * RPS
    nfull = RPS // CH
    rem = RPS - nfull * CH
    for j in range(nfull):
        pltpu.sync_copy(gbuf1, acc.at[pl.ds(row0 + j * CH, CH)])
    pltpu.sync_copy(gbuf1.at[pl.ds(0, rem)],
                    acc.at[pl.ds(row0 + nfull * CH, rem)])

    @pl.when(s == NS - 1)
    def _():
        pltpu.sync_copy(gbuf1.at[pl.ds(0, RTAIL)],
                        acc.at[pl.ds(NS * RPS, RTAIL)])

    plsc.subcore_barrier()

    def chunk_body(g, k):
        b = k % 2
        sidx, sidxo = (sidx0, sidx1) if b == 0 else (sidx1, sidx0)
        isem, isemo = (isem0, isem1) if b == 0 else (isem1, isem0)
        didxb = didx0 if b == 0 else didx1
        dsb = dsem0 if b == 0 else dsem1
        mb = mbuf0 if b == 0 else mbuf1
        gb, gbo = (gbuf0, gbuf1) if b == 0 else (gbuf1, gbuf0)
        easem = easem0 if b == 0 else easem1
        gsem, gsemo = (gsem0, gsem1) if b == 0 else (gsem1, gsem0)

        pltpu.make_async_copy(ea_hbm.at[pl.ds(ebase + g * CH, CH)],
                              mb, easem).wait()
        pltpu.make_async_copy(x_hbm.at[sidx], gb, gsem).wait()

        @pl.when(g + 2 < NCHUNK)
        def _():
            pltpu.async_copy(src_hbm.at[pl.ds(ebase + (g + 2) * CH, CH)], sidx, isem)

        @pl.when(g + 1 < NCHUNK)
        def _():
            pltpu.make_async_copy(src_hbm.at[pl.ds(ebase + (g + 1) * CH, CH)], sidxo, isemo).wait()
            pltpu.async_copy(x_hbm.at[sidxo], gbo, gsemo)

        @plsc.parallel_loop(0, CH, 1, unroll=4)
        def _(r):
            for j in range(D // L):
                sl = (r, pl.ds(j * L, L))
                gb[sl] = jnp.maximum(mb[sl] + gb[sl], 0.0)

        pltpu.make_async_copy(dst_hbm.at[pl.ds(ebase + g * CH, CH)], didxb, dsb).wait()
        pltpu.sync_copy(gb, acc.at[didxb], add=True)

        @pl.when(g + 2 < NCHUNK)
        def _():
            pltpu.async_copy(dst_hbm.at[pl.ds(ebase + (g + 2) * CH, CH)], didxb, dsb)
            pltpu.async_copy(ea_hbm.at[pl.ds(ebase + (g + 2) * CH, CH)],
                             mb, easem)

    @pl.loop(0, NCHUNK - 1, step=2)
    def _(g):
        chunk_body(g, 0)
        chunk_body(g + 1, 1)

    chunk_body(jnp.int32(NCHUNK - 1), 0)

    plsc.subcore_barrier()

    pltpu.sync_copy(acc.at[pl.ds(row0, RPS)], out_hbm.at[c, pl.ds(row0, RPS)])

    @pl.when(s == NS - 1)
    def _():
        pltpu.sync_copy(acc.at[pl.ds(NS * RPS, RTAIL)],
                        out_hbm.at[c, pl.ds(NS * RPS, RTAIL)])


def _sc_aggr(x, src2, dst2, edge_attr):
    mesh = plsc.VectorSubcoreMesh(core_axis_name="c", subcore_axis_name="s")
    k = pl.kernel(
        _sc_aggr_kernel,
        out_type=jax.ShapeDtypeStruct((NC, N, D), jnp.float32),
        mesh=mesh,
        scratch_types=[
            pltpu.VMEM((CH,), jnp.int32),
            pltpu.VMEM((CH,), jnp.int32),
            pltpu.VMEM((CH,), jnp.int32),
            pltpu.VMEM((CH,), jnp.int32),
            pltpu.VMEM((CH, D), jnp.float32),
            pltpu.VMEM((CH, D), jnp.float32),
            pltpu.VMEM((CH, D), jnp.float32),
            pltpu.VMEM((CH, D), jnp.float32),
            pltpu.VMEM_SHARED((N, D), jnp.float32),
            pltpu.SemaphoreType.DMA,
            pltpu.SemaphoreType.DMA,
            pltpu.SemaphoreType.DMA,
            pltpu.SemaphoreType.DMA,
            pltpu.SemaphoreType.DMA,
            pltpu.SemaphoreType.DMA,
            pltpu.SemaphoreType.DMA,
            pltpu.SemaphoreType.DMA,
        ],
    )
    return k(x, src2, dst2, edge_attr)


def _tc_body(x_ref, p0_ref, p1_ref, w1_ref, b1_ref, w2_ref, b2_ref,
             eps_ref, g_ref, bt_ref, o_ref):
    z = x_ref[...] * (1.0 + eps_ref[0, 0]) + p0_ref[...] + p1_ref[...]
    h = jnp.dot(z, w1_ref[...], preferred_element_type=jnp.float32) + b1_ref[...]
    h = jnp.maximum(h, 0.0)
    h = jnp.dot(h, w2_ref[...], preferred_element_type=jnp.float32) + b2_ref[...]
    mean = jnp.mean(h, axis=1, keepdims=True)
    hc = h - mean
    var = jnp.mean(hc * hc, axis=1, keepdims=True)
    hn = hc * lax.rsqrt(var + 1e-5) * g_ref[...] + bt_ref[...]
    o_ref[...] = jnp.maximum(hn, 0.0)


BLK = 1000


def _tc_mlp(x, p0, p1, W1, b1, W2, b2, eps11, gamma, beta):
    grid = (N // BLK,)
    row_spec = pl.BlockSpec((BLK, D), lambda i: (i, 0))
    full_spec = pl.BlockSpec((D, D), lambda i: (0, 0))
    vec_spec = pl.BlockSpec((1, D), lambda i: (0, 0))
    return pl.pallas_call(
        _tc_body,
        grid=grid,
        in_specs=[row_spec, row_spec, row_spec,
                  full_spec, vec_spec, full_spec, vec_spec,
                  pl.BlockSpec((1, 1), lambda i: (0, 0)),
                  vec_spec, vec_spec],
        out_specs=row_spec,
        out_shape=jax.ShapeDtypeStruct((N, D), jnp.float32),
    )(x, p0, p1, W1, b1, W2, b2, eps11, gamma, beta)


def kernel(x, edge_index, edge_attr, W1, b1, W2, b2, eps, gamma, beta):
    parts = _sc_aggr(x, edge_index[0], edge_index[1], edge_attr)
    eps11 = jnp.reshape(eps, (1, 1)).astype(jnp.float32)
    return _tc_mlp(x, parts[0], parts[1], W1,
                   jnp.reshape(b1, (1, D)), W2, jnp.reshape(b2, (1, D)),
                   eps11, jnp.reshape(gamma, (1, D)), jnp.reshape(beta, (1, D)))


# speedup vs baseline: 1.4411x; 1.1164x over previous
import jax
import jax.numpy as jnp
from jax import lax
from jax.experimental import pallas as pl
from jax.experimental.pallas import tpu as pltpu
from jax.experimental.pallas import tpu_sc as plsc

N = 10000
E = 320000
D = 128
L = 16
NC = 2
NS = 16
NW = NC * NS
EPW = E // NW
CH = 64
NCHUNK = 156
T = EPW - NCHUNK * CH
RPS = 624
RTAIL = N - NS * RPS


def _sc_aggr_kernel(x_hbm, ei_hbm, ea_hbm, out_hbm,
                    sidx0, sidx1, sidxt, didx0, didx1, didx2, didx3, didxt,
                    mbuf0, mbuf1, gbuf0, gbuf1, sbuf0, sbuf1, acc,
                    isem0, isem1, dsem0, dsem1, dsem2, dsem3,
                    easem0, easem1, gsem0, gsem1, scsem0, scsem1):
    c = lax.axis_index("c")
    s = lax.axis_index("s")
    wid = c * NS + s
    ebase = wid * EPW
    dbase = E + wid * EPW
    didx = (didx0, didx1, didx2, didx3)
    dsem = (dsem0, dsem1, dsem2, dsem3)

    for d in range(4):
        pltpu.async_copy(ei_hbm.at[pl.ds(dbase + d * CH, CH)],
                         didx[d], dsem[d])
    pltpu.async_copy(ei_hbm.at[pl.ds(ebase, CH)], sidx0, isem0)
    pltpu.async_copy(ei_hbm.at[pl.ds(ebase + CH, CH)], sidx1, isem1)
    pltpu.async_copy(ea_hbm.at[pl.ds(ebase, CH)], mbuf0, easem0)
    pltpu.async_copy(ea_hbm.at[pl.ds(ebase + CH, CH)], mbuf1, easem1)
    pltpu.make_async_copy(ei_hbm.at[pl.ds(ebase, CH)], sidx0, isem0).wait()
    pltpu.async_copy(x_hbm.at[sidx0], gbuf0, gsem0)

    @pl.loop(0, CH)
    def _(r):
        for j in range(D // L):
            sbuf0[r, pl.ds(j * L, L)] = jnp.zeros((L,), jnp.float32)

    row0 = s * RPS
    nfull = RPS // CH
    rem = RPS - nfull * CH
    for j in range(nfull):
        pltpu.sync_copy(sbuf0, acc.at[pl.ds(row0 + j * CH, CH)])
    pltpu.sync_copy(sbuf0.at[pl.ds(0, rem)],
                    acc.at[pl.ds(row0 + nfull * CH, rem)])

    @pl.when(s == NS - 1)
    def _():
        pltpu.sync_copy(sbuf0.at[pl.ds(0, RTAIL)],
                        acc.at[pl.ds(NS * RPS, RTAIL)])

    plsc.subcore_barrier()

    def chunk_body(g, k):
        b = k % 2
        q, q2 = k % 4, (k + 2) % 4
        sidx, sidxo = (sidx0, sidx1) if b == 0 else (sidx1, sidx0)
        isem, isemo = (isem0, isem1) if b == 0 else (isem1, isem0)
        mb = mbuf0 if b == 0 else mbuf1
        gb, gbo = (gbuf0, gbuf1) if b == 0 else (gbuf1, gbuf0)
        sb = sbuf0 if b == 0 else sbuf1
        easem = easem0 if b == 0 else easem1
        gsem, gsemo = (gsem0, gsem1) if b == 0 else (gsem1, gsem0)
        scsem = scsem0 if b == 0 else scsem1

        pltpu.make_async_copy(ea_hbm.at[pl.ds(ebase + g * CH, CH)],
                              mb, easem).wait()
        pltpu.make_async_copy(x_hbm.at[sidx], gb, gsem).wait()

        @pl.when(g + 2 < NCHUNK)
        def _():
            pltpu.async_copy(ei_hbm.at[pl.ds(ebase + (g + 2) * CH, CH)],
                             sidx, isem)

        @pl.when(g + 1 < NCHUNK)
        def _():
            pltpu.make_async_copy(ei_hbm.at[pl.ds(ebase + (g + 1) * CH, CH)],
                                  sidxo, isemo).wait()
            pltpu.async_copy(x_hbm.at[sidxo], gbo, gsemo)

        @pl.when(g >= 2)
        def _():
            pltpu.make_async_copy(sb, acc.at[didx[q2]], scsem).wait()

            @pl.when(g + 2 < NCHUNK)
            def _():
                pltpu.async_copy(ei_hbm.at[pl.ds(dbase + (g + 2) * CH, CH)],
                                 didx[q2], dsem[q2])

        @plsc.parallel_loop(0, CH, 1, unroll=4)
        def _(r):
            for j in range(D // L):
                sl = (r, pl.ds(j * L, L))
                sb[sl] = jnp.maximum(mb[sl] + gb[sl], 0.0)

        pltpu.make_async_copy(ei_hbm.at[pl.ds(dbase + g * CH, CH)],
                              didx[q], dsem[q]).wait()
        pltpu.async_copy(sb, acc.at[didx[q]], scsem, add=True)

        @pl.when(g + 2 < NCHUNK)
        def _():
            pltpu.async_copy(ea_hbm.at[pl.ds(ebase + (g + 2) * CH, CH)],
                             mb, easem)

    @pl.loop(0, NCHUNK, step=4)
    def _(g):
        for k in range(4):
            chunk_body(g + k, k)

    tbase = ebase + NCHUNK * CH
    pltpu.sync_copy(ei_hbm.at[pl.ds(tbase, T)], sidxt)
    pltpu.sync_copy(ei_hbm.at[pl.ds(E + tbase, T)], didxt)
    pltpu.sync_copy(ea_hbm.at[pl.ds(tbase, T)], mbuf0.at[pl.ds(0, T)])
    pltpu.sync_copy(x_hbm.at[sidxt], gbuf0.at[pl.ds(0, T)])
    pltpu.make_async_copy(sbuf0, acc.at[didx0], scsem0).wait()

    @plsc.parallel_loop(0, T, 1, unroll=4)
    def _(r):
        for j in range(D // L):
            sl = (r, pl.ds(j * L, L))
            sbuf0[sl] = jnp.maximum(mbuf0[sl] + gbuf0[sl], 0.0)

    pltpu.sync_copy(sbuf0.at[pl.ds(0, T)], acc.at[didxt], add=True)
    pltpu.make_async_copy(sbuf1, acc.at[didx1], scsem1).wait()

    plsc.subcore_barrier()

    pltpu.sync_copy(acc.at[pl.ds(row0, RPS)], out_hbm.at[c, pl.ds(row0, RPS)])

    @pl.when(s == NS - 1)
    def _():
        pltpu.sync_copy(acc.at[pl.ds(NS * RPS, RTAIL)],
                        out_hbm.at[c, pl.ds(NS * RPS, RTAIL)])


def _sc_aggr(x, ei_flat, edge_attr):
    mesh = plsc.VectorSubcoreMesh(core_axis_name="c", subcore_axis_name="s")
    k = pl.kernel(
        _sc_aggr_kernel,
        out_type=jax.ShapeDtypeStruct((NC, N, D), jnp.float32),
        mesh=mesh,
        scratch_types=[
            pltpu.VMEM((CH,), jnp.int32),
            pltpu.VMEM((CH,), jnp.int32),
            pltpu.VMEM((T,), jnp.int32),
            pltpu.VMEM((CH,), jnp.int32),
            pltpu.VMEM((CH,), jnp.int32),
            pltpu.VMEM((CH,), jnp.int32),
            pltpu.VMEM((CH,), jnp.int32),
            pltpu.VMEM((T,), jnp.int32),
            pltpu.VMEM((CH, D), jnp.float32),
            pltpu.VMEM((CH, D), jnp.float32),
            pltpu.VMEM((CH, D), jnp.float32),
            pltpu.VMEM((CH, D), jnp.float32),
            pltpu.VMEM((CH, D), jnp.float32),
            pltpu.VMEM((CH, D), jnp.float32),
            pltpu.VMEM_SHARED((N, D), jnp.float32),
            pltpu.SemaphoreType.DMA,
            pltpu.SemaphoreType.DMA,
            pltpu.SemaphoreType.DMA,
            pltpu.SemaphoreType.DMA,
            pltpu.SemaphoreType.DMA,
            pltpu.SemaphoreType.DMA,
            pltpu.SemaphoreType.DMA,
            pltpu.SemaphoreType.DMA,
            pltpu.SemaphoreType.DMA,
            pltpu.SemaphoreType.DMA,
            pltpu.SemaphoreType.DMA,
            pltpu.SemaphoreType.DMA,
        ],
    )
    return k(x, ei_flat, edge_attr)


def _tc_body(x_ref, p_ref, w1_ref, b1_ref, w2_ref, b2_ref,
             eps_ref, g_ref, bt_ref, o_ref):
    z = x_ref[...] * (1.0 + eps_ref[0, 0]) + p_ref[0] + p_ref[1]
    h = jnp.dot(z, w1_ref[...], preferred_element_type=jnp.float32) + b1_ref[...]
    h = jnp.maximum(h, 0.0)
    h = jnp.dot(h, w2_ref[...], preferred_element_type=jnp.float32) + b2_ref[...]
    mean = jnp.mean(h, axis=1, keepdims=True)
    hc = h - mean
    var = jnp.mean(hc * hc, axis=1, keepdims=True)
    hn = hc * lax.rsqrt(var + 1e-5) * g_ref[...] + bt_ref[...]
    o_ref[...] = jnp.maximum(hn, 0.0)


BLK = 2000


def _tc_mlp(x, parts, W1, b1, W2, b2, eps11, gamma, beta):
    grid = (N // BLK,)
    row_spec = pl.BlockSpec((BLK, D), lambda i: (i, 0))
    full_spec = pl.BlockSpec((D, D), lambda i: (0, 0))
    vec_spec = pl.BlockSpec((1, D), lambda i: (0, 0))
    return pl.pallas_call(
        _tc_body,
        grid=grid,
        in_specs=[row_spec,
                  pl.BlockSpec((NC, BLK, D), lambda i: (0, i, 0)),
                  full_spec, vec_spec, full_spec, vec_spec,
                  pl.BlockSpec((1, 1), lambda i: (0, 0)),
                  vec_spec, vec_spec],
        out_specs=row_spec,
        out_shape=jax.ShapeDtypeStruct((N, D), jnp.float32),
    )(x, parts, W1, b1, W2, b2, eps11, gamma, beta)


def kernel(x, edge_index, edge_attr, W1, b1, W2, b2, eps, gamma, beta):
    ei_flat = edge_index.reshape(2 * E)
    parts = _sc_aggr(x, ei_flat, edge_attr)
    eps11 = jnp.reshape(eps, (1, 1)).astype(jnp.float32)
    return _tc_mlp(x, parts, W1,
                   jnp.reshape(b1, (1, D)), W2, jnp.reshape(b2, (1, D)),
                   eps11, jnp.reshape(gamma, (1, D)), jnp.reshape(beta, (1, D)))


# speedup vs baseline: 1.4435x; 1.0017x over previous
import jax
import jax.numpy as jnp
from jax import lax
from jax.experimental import pallas as pl
from jax.experimental.pallas import tpu as pltpu
from jax.experimental.pallas import tpu_sc as plsc

N = 10000
E = 320000
D = 128
L = 16
NC = 2
NS = 16
NW = NC * NS
EPW = E // NW
CH = 64
NCHUNK = 156
T = EPW - NCHUNK * CH
RPS = 624
RTAIL = N - NS * RPS


def _sc_aggr_kernel(x_hbm, ei_hbm, ea_hbm, out_hbm,
                    sidx0, sidx1, sidxt, didx0, didx1, didx2, didx3, didxt,
                    mbuf0, mbuf1, gbuf0, gbuf1, sbuf0, sbuf1, acc,
                    isem0, isem1, dsem0, dsem1, dsem2, dsem3,
                    easem0, easem1, gsem0, gsem1, scsem0, scsem1):
    c = lax.axis_index("c")
    s = lax.axis_index("s")
    wid = c * NS + s
    ebase = wid * EPW
    dbase = E + wid * EPW
    didx = (didx0, didx1, didx2, didx3)
    dsem = (dsem0, dsem1, dsem2, dsem3)

    for d in range(4):
        pltpu.async_copy(ei_hbm.at[pl.ds(dbase + d * CH, CH)],
                         didx[d], dsem[d])
    pltpu.async_copy(ei_hbm.at[pl.ds(ebase, CH)], sidx0, isem0)
    pltpu.async_copy(ei_hbm.at[pl.ds(ebase + CH, CH)], sidx1, isem1)
    pltpu.async_copy(ea_hbm.at[pl.ds(ebase, CH)], mbuf0, easem0)
    pltpu.async_copy(ea_hbm.at[pl.ds(ebase + CH, CH)], mbuf1, easem1)
    pltpu.make_async_copy(ei_hbm.at[pl.ds(ebase, CH)], sidx0, isem0).wait()
    pltpu.async_copy(x_hbm.at[sidx0.at[pl.ds(0, CH // 2)]],
                     gbuf0.at[pl.ds(0, CH // 2)], gsem0)
    pltpu.async_copy(x_hbm.at[sidx0.at[pl.ds(CH // 2, CH // 2)]],
                     gbuf0.at[pl.ds(CH // 2, CH // 2)], gsem0)

    @pl.loop(0, CH)
    def _(r):
        for j in range(D // L):
            sbuf0[r, pl.ds(j * L, L)] = jnp.zeros((L,), jnp.float32)

    row0 = s * RPS
    nfull = RPS // CH
    rem = RPS - nfull * CH
    for j in range(nfull):
        pltpu.sync_copy(sbuf0, acc.at[pl.ds(row0 + j * CH, CH)])
    pltpu.sync_copy(sbuf0.at[pl.ds(0, rem)],
                    acc.at[pl.ds(row0 + nfull * CH, rem)])

    @pl.when(s == NS - 1)
    def _():
        pltpu.sync_copy(sbuf0.at[pl.ds(0, RTAIL)],
                        acc.at[pl.ds(NS * RPS, RTAIL)])

    plsc.subcore_barrier()

    def chunk_body(g, k):
        b = k % 2
        q, q2 = k % 4, (k + 2) % 4
        sidx, sidxo = (sidx0, sidx1) if b == 0 else (sidx1, sidx0)
        isem, isemo = (isem0, isem1) if b == 0 else (isem1, isem0)
        mb = mbuf0 if b == 0 else mbuf1
        gb, gbo = (gbuf0, gbuf1) if b == 0 else (gbuf1, gbuf0)
        sb = sbuf0 if b == 0 else sbuf1
        easem = easem0 if b == 0 else easem1
        gsem, gsemo = (gsem0, gsem1) if b == 0 else (gsem1, gsem0)
        scsem = scsem0 if b == 0 else scsem1

        pltpu.make_async_copy(ea_hbm.at[pl.ds(ebase + g * CH, CH)],
                              mb, easem).wait()
        pltpu.make_async_copy(x_hbm.at[sidx], gb, gsem).wait()

        @pl.when(g + 2 < NCHUNK)
        def _():
            pltpu.async_copy(ei_hbm.at[pl.ds(ebase + (g + 2) * CH, CH)],
                             sidx, isem)

        @pl.when(g + 1 < NCHUNK)
        def _():
            pltpu.make_async_copy(ei_hbm.at[pl.ds(ebase + (g + 1) * CH, CH)],
                                  sidxo, isemo).wait()
            pltpu.async_copy(x_hbm.at[sidxo.at[pl.ds(0, CH // 2)]],
                             gbo.at[pl.ds(0, CH // 2)], gsemo)
            pltpu.async_copy(x_hbm.at[sidxo.at[pl.ds(CH // 2, CH // 2)]],
                             gbo.at[pl.ds(CH // 2, CH // 2)], gsemo)

        @pl.when(g >= 2)
        def _():
            pltpu.make_async_copy(sb, acc.at[didx[q2]], scsem).wait()

            @pl.when(g + 2 < NCHUNK)
            def _():
                pltpu.async_copy(ei_hbm.at[pl.ds(dbase + (g + 2) * CH, CH)],
                                 didx[q2], dsem[q2])

        @plsc.parallel_loop(0, CH, 1, unroll=4)
        def _(r):
            for j in range(D // L):
                sl = (r, pl.ds(j * L, L))
                sb[sl] = jnp.maximum(mb[sl] + gb[sl], 0.0)

        pltpu.make_async_copy(ei_hbm.at[pl.ds(dbase + g * CH, CH)],
                              didx[q], dsem[q]).wait()
        pltpu.async_copy(sb, acc.at[didx[q]], scsem, add=True)

        @pl.when(g + 2 < NCHUNK)
        def _():
            pltpu.async_copy(ea_hbm.at[pl.ds(ebase + (g + 2) * CH, CH)],
                             mb, easem)

    @pl.loop(0, NCHUNK, step=4)
    def _(g):
        for k in range(4):
            chunk_body(g + k, k)

    tbase = ebase + NCHUNK * CH
    pltpu.sync_copy(ei_hbm.at[pl.ds(tbase, T)], sidxt)
    pltpu.sync_copy(ei_hbm.at[pl.ds(E + tbase, T)], didxt)
    pltpu.sync_copy(ea_hbm.at[pl.ds(tbase, T)], mbuf0.at[pl.ds(0, T)])
    pltpu.sync_copy(x_hbm.at[sidxt], gbuf0.at[pl.ds(0, T)])
    pltpu.make_async_copy(sbuf0, acc.at[didx0], scsem0).wait()

    @plsc.parallel_loop(0, T, 1, unroll=4)
    def _(r):
        for j in range(D // L):
            sl = (r, pl.ds(j * L, L))
            sbuf0[sl] = jnp.maximum(mbuf0[sl] + gbuf0[sl], 0.0)

    pltpu.sync_copy(sbuf0.at[pl.ds(0, T)], acc.at[didxt], add=True)
    pltpu.make_async_copy(sbuf1, acc.at[didx1], scsem1).wait()

    plsc.subcore_barrier()

    pltpu.sync_copy(acc.at[pl.ds(row0, RPS)], out_hbm.at[c, pl.ds(row0, RPS)])

    @pl.when(s == NS - 1)
    def _():
        pltpu.sync_copy(acc.at[pl.ds(NS * RPS, RTAIL)],
                        out_hbm.at[c, pl.ds(NS * RPS, RTAIL)])


def _sc_aggr(x, ei_flat, edge_attr):
    mesh = plsc.VectorSubcoreMesh(core_axis_name="c", subcore_axis_name="s")
    k = pl.kernel(
        _sc_aggr_kernel,
        out_type=jax.ShapeDtypeStruct((NC, N, D), jnp.float32),
        mesh=mesh,
        scratch_types=[
            pltpu.VMEM((CH,), jnp.int32),
            pltpu.VMEM((CH,), jnp.int32),
            pltpu.VMEM((T,), jnp.int32),
            pltpu.VMEM((CH,), jnp.int32),
            pltpu.VMEM((CH,), jnp.int32),
            pltpu.VMEM((CH,), jnp.int32),
            pltpu.VMEM((CH,), jnp.int32),
            pltpu.VMEM((T,), jnp.int32),
            pltpu.VMEM((CH, D), jnp.float32),
            pltpu.VMEM((CH, D), jnp.float32),
            pltpu.VMEM((CH, D), jnp.float32),
            pltpu.VMEM((CH, D), jnp.float32),
            pltpu.VMEM((CH, D), jnp.float32),
            pltpu.VMEM((CH, D), jnp.float32),
            pltpu.VMEM_SHARED((N, D), jnp.float32),
            pltpu.SemaphoreType.DMA,
            pltpu.SemaphoreType.DMA,
            pltpu.SemaphoreType.DMA,
            pltpu.SemaphoreType.DMA,
            pltpu.SemaphoreType.DMA,
            pltpu.SemaphoreType.DMA,
            pltpu.SemaphoreType.DMA,
            pltpu.SemaphoreType.DMA,
            pltpu.SemaphoreType.DMA,
            pltpu.SemaphoreType.DMA,
            pltpu.SemaphoreType.DMA,
            pltpu.SemaphoreType.DMA,
        ],
    )
    return k(x, ei_flat, edge_attr)


def _tc_body(x_ref, p_ref, w1_ref, b1_ref, w2_ref, b2_ref,
             eps_ref, g_ref, bt_ref, o_ref):
    z = x_ref[...] * (1.0 + eps_ref[0, 0]) + p_ref[0] + p_ref[1]
    h = jnp.dot(z, w1_ref[...], preferred_element_type=jnp.float32) + b1_ref[...]
    h = jnp.maximum(h, 0.0)
    h = jnp.dot(h, w2_ref[...], preferred_element_type=jnp.float32) + b2_ref[...]
    mean = jnp.mean(h, axis=1, keepdims=True)
    hc = h - mean
    var = jnp.mean(hc * hc, axis=1, keepdims=True)
    hn = hc * lax.rsqrt(var + 1e-5) * g_ref[...] + bt_ref[...]
    o_ref[...] = jnp.maximum(hn, 0.0)


BLK = 2000


def _tc_mlp(x, parts, W1, b1, W2, b2, eps11, gamma, beta):
    grid = (N // BLK,)
    row_spec = pl.BlockSpec((BLK, D), lambda i: (i, 0))
    full_spec = pl.BlockSpec((D, D), lambda i: (0, 0))
    vec_spec = pl.BlockSpec((1, D), lambda i: (0, 0))
    return pl.pallas_call(
        _tc_body,
        grid=grid,
        in_specs=[row_spec,
                  pl.BlockSpec((NC, BLK, D), lambda i: (0, i, 0)),
                  full_spec, vec_spec, full_spec, vec_spec,
                  pl.BlockSpec((1, 1), lambda i: (0, 0)),
                  vec_spec, vec_spec],
        out_specs=row_spec,
        out_shape=jax.ShapeDtypeStruct((N, D), jnp.float32),
    )(x, parts, W1, b1, W2, b2, eps11, gamma, beta)


def kernel(x, edge_index, edge_attr, W1, b1, W2, b2, eps, gamma, beta):
    ei_flat = edge_index.reshape(2 * E)
    parts = _sc_aggr(x, ei_flat, edge_attr)
    eps11 = jnp.reshape(eps, (1, 1)).astype(jnp.float32)
    return _tc_mlp(x, parts, W1,
                   jnp.reshape(b1, (1, D)), W2, jnp.reshape(b2, (1, D)),
                   eps11, jnp.reshape(gamma, (1, D)), jnp.reshape(beta, (1, D)))


# speedup vs baseline: 1.4450x; 1.0010x over previous
import jax
import jax.numpy as jnp
from jax import lax
from jax.experimental import pallas as pl
from jax.experimental.pallas import tpu as pltpu
from jax.experimental.pallas import tpu_sc as plsc

N = 10000
E = 320000
D = 128
L = 16
NC = 2
NS = 16
NW = NC * NS
EPW = E // NW
CH = 64
NCHUNK = 156
T = EPW - NCHUNK * CH
RPS = 624
RTAIL = N - NS * RPS


def _sc_aggr_kernel(x_hbm, ei_hbm, ea_hbm, out_hbm,
                    sidx0, sidx1, sidxt, didx0, didx1, didx2, didx3, didxt,
                    mbuf0, mbuf1, gbuf0, gbuf1, sbuf0, sbuf1, acc,
                    isem0, isem1, dsem0, dsem1, dsem2, dsem3,
                    easem0, easem1, gsem0, gsem1, scsem0, scsem1):
    c = lax.axis_index("c")
    s = lax.axis_index("s")
    wid = c * NS + s
    ebase = wid * EPW
    dbase = E + wid * EPW
    didx = (didx0, didx1, didx2, didx3)
    dsem = (dsem0, dsem1, dsem2, dsem3)

    for d in range(4):
        pltpu.async_copy(ei_hbm.at[pl.ds(dbase + d * CH, CH)],
                         didx[d], dsem[d])
    pltpu.async_copy(ei_hbm.at[pl.ds(ebase, CH)], sidx0, isem0)
    pltpu.async_copy(ei_hbm.at[pl.ds(ebase + CH, CH)], sidx1, isem1)
    pltpu.async_copy(ea_hbm.at[pl.ds(ebase, CH)], mbuf0, easem0)
    pltpu.async_copy(ea_hbm.at[pl.ds(ebase + CH, CH)], mbuf1, easem1)
    pltpu.make_async_copy(ei_hbm.at[pl.ds(ebase, CH)], sidx0, isem0).wait()
    pltpu.async_copy(x_hbm.at[sidx0], gbuf0, gsem0)

    @pl.loop(0, CH)
    def _(r):
        for j in range(D // L):
            sbuf0[r, pl.ds(j * L, L)] = jnp.zeros((L,), jnp.float32)

    row0 = s * RPS
    nfull = RPS // CH
    rem = RPS - nfull * CH
    for j in range(nfull):
        pltpu.sync_copy(sbuf0, acc.at[pl.ds(row0 + j * CH, CH)])
    pltpu.sync_copy(sbuf0.at[pl.ds(0, rem)],
                    acc.at[pl.ds(row0 + nfull * CH, rem)])

    @pl.when(s == NS - 1)
    def _():
        pltpu.sync_copy(sbuf0.at[pl.ds(0, RTAIL)],
                        acc.at[pl.ds(NS * RPS, RTAIL)])

    plsc.subcore_barrier()

    def chunk_body(g, k):
        b = k % 2
        q, q2 = k % 4, (k + 2) % 4
        sidx, sidxo = (sidx0, sidx1) if b == 0 else (sidx1, sidx0)
        isem, isemo = (isem0, isem1) if b == 0 else (isem1, isem0)
        mb = mbuf0 if b == 0 else mbuf1
        gb, gbo = (gbuf0, gbuf1) if b == 0 else (gbuf1, gbuf0)
        sb = sbuf0 if b == 0 else sbuf1
        easem = easem0 if b == 0 else easem1
        gsem, gsemo = (gsem0, gsem1) if b == 0 else (gsem1, gsem0)
        scsem = scsem0 if b == 0 else scsem1

        pltpu.make_async_copy(ea_hbm.at[pl.ds(ebase + g * CH, CH)],
                              mb, easem).wait()
        pltpu.make_async_copy(x_hbm.at[sidx], gb, gsem).wait()

        @pl.when(g + 2 < NCHUNK)
        def _():
            pltpu.async_copy(ei_hbm.at[pl.ds(ebase + (g + 2) * CH, CH)],
                             sidx, isem)

        @pl.when(g + 1 < NCHUNK)
        def _():
            pltpu.make_async_copy(ei_hbm.at[pl.ds(ebase + (g + 1) * CH, CH)],
                                  sidxo, isemo).wait()
            pltpu.async_copy(x_hbm.at[sidxo], gbo, gsemo)

        @pl.when(g >= 2)
        def _():
            pltpu.make_async_copy(sb, acc.at[didx[q2]], scsem).wait()

            @pl.when(g + 2 < NCHUNK)
            def _():
                pltpu.async_copy(ei_hbm.at[pl.ds(dbase + (g + 2) * CH, CH)],
                                 didx[q2], dsem[q2])

        @plsc.parallel_loop(0, CH, 1, unroll=4)
        def _(r):
            for j in range(D // L):
                sl = (r, pl.ds(j * L, L))
                sb[sl] = jnp.maximum(mb[sl] + gb[sl], 0.0)

        pltpu.make_async_copy(ei_hbm.at[pl.ds(dbase + g * CH, CH)],
                              didx[q], dsem[q]).wait()
        pltpu.async_copy(sb, acc.at[didx[q]], scsem, add=True)

        @pl.when(g + 2 < NCHUNK)
        def _():
            pltpu.async_copy(ea_hbm.at[pl.ds(ebase + (g + 2) * CH, CH)],
                             mb, easem)

    @pl.loop(0, NCHUNK, step=4)
    def _(g):
        for k in range(4):
            chunk_body(g + k, k)

    tbase = ebase + NCHUNK * CH
    pltpu.sync_copy(ei_hbm.at[pl.ds(tbase, T)], sidxt)
    pltpu.sync_copy(ei_hbm.at[pl.ds(E + tbase, T)], didxt)
    pltpu.sync_copy(ea_hbm.at[pl.ds(tbase, T)], mbuf0.at[pl.ds(0, T)])
    pltpu.sync_copy(x_hbm.at[sidxt], gbuf0.at[pl.ds(0, T)])
    pltpu.make_async_copy(sbuf0, acc.at[didx0], scsem0).wait()

    @plsc.parallel_loop(0, T, 1, unroll=4)
    def _(r):
        for j in range(D // L):
            sl = (r, pl.ds(j * L, L))
            sbuf0[sl] = jnp.maximum(mbuf0[sl] + gbuf0[sl], 0.0)

    pltpu.sync_copy(sbuf0.at[pl.ds(0, T)], acc.at[didxt], add=True)
    pltpu.make_async_copy(sbuf1, acc.at[didx1], scsem1).wait()

    plsc.subcore_barrier()

    pltpu.sync_copy(acc.at[pl.ds(row0, RPS)], out_hbm.at[c, pl.ds(row0, RPS)])

    @pl.when(s == NS - 1)
    def _():
        pltpu.sync_copy(acc.at[pl.ds(NS * RPS, RTAIL)],
                        out_hbm.at[c, pl.ds(NS * RPS, RTAIL)])


def _sc_aggr(x, ei_flat, edge_attr):
    mesh = plsc.VectorSubcoreMesh(core_axis_name="c", subcore_axis_name="s")
    k = pl.kernel(
        _sc_aggr_kernel,
        out_type=jax.ShapeDtypeStruct((NC, N, D), jnp.float32),
        mesh=mesh,
        scratch_types=[
            pltpu.VMEM((CH,), jnp.int32),
            pltpu.VMEM((CH,), jnp.int32),
            pltpu.VMEM((T,), jnp.int32),
            pltpu.VMEM((CH,), jnp.int32),
            pltpu.VMEM((CH,), jnp.int32),
            pltpu.VMEM((CH,), jnp.int32),
            pltpu.VMEM((CH,), jnp.int32),
            pltpu.VMEM((T,), jnp.int32),
            pltpu.VMEM((CH, D), jnp.float32),
            pltpu.VMEM((CH, D), jnp.float32),
            pltpu.VMEM((CH, D), jnp.float32),
            pltpu.VMEM((CH, D), jnp.float32),
            pltpu.VMEM((CH, D), jnp.float32),
            pltpu.VMEM((CH, D), jnp.float32),
            pltpu.VMEM_SHARED((N, D), jnp.float32),
            pltpu.SemaphoreType.DMA,
            pltpu.SemaphoreType.DMA,
            pltpu.SemaphoreType.DMA,
            pltpu.SemaphoreType.DMA,
            pltpu.SemaphoreType.DMA,
            pltpu.SemaphoreType.DMA,
            pltpu.SemaphoreType.DMA,
            pltpu.SemaphoreType.DMA,
            pltpu.SemaphoreType.DMA,
            pltpu.SemaphoreType.DMA,
            pltpu.SemaphoreType.DMA,
            pltpu.SemaphoreType.DMA,
        ],
    )
    return k(x, ei_flat, edge_attr)


def _tc_body(x_ref, p_ref, w1_ref, b1_ref, w2_ref, b2_ref,
             eps_ref, g_ref, bt_ref, o_ref):
    z = x_ref[...] * (1.0 + eps_ref[0, 0]) + p_ref[0] + p_ref[1]
    h = jnp.dot(z, w1_ref[...], preferred_element_type=jnp.float32) + b1_ref[...]
    h = jnp.maximum(h, 0.0)
    h = jnp.dot(h, w2_ref[...], preferred_element_type=jnp.float32) + b2_ref[...]
    mean = jnp.mean(h, axis=1, keepdims=True)
    hc = h - mean
    var = jnp.mean(hc * hc, axis=1, keepdims=True)
    hn = hc * lax.rsqrt(var + 1e-5) * g_ref[...] + bt_ref[...]
    o_ref[...] = jnp.maximum(hn, 0.0)


BLK = 2000


def _tc_mlp(x, parts, W1, b1, W2, b2, eps11, gamma, beta):
    grid = (N // BLK,)
    row_spec = pl.BlockSpec((BLK, D), lambda i: (i, 0))
    full_spec = pl.BlockSpec((D, D), lambda i: (0, 0))
    vec_spec = pl.BlockSpec((1, D), lambda i: (0, 0))
    return pl.pallas_call(
        _tc_body,
        grid=grid,
        in_specs=[row_spec,
                  pl.BlockSpec((NC, BLK, D), lambda i: (0, i, 0)),
                  full_spec, vec_spec, full_spec, vec_spec,
                  pl.BlockSpec((1, 1), lambda i: (0, 0)),
                  vec_spec, vec_spec],
        out_specs=row_spec,
        out_shape=jax.ShapeDtypeStruct((N, D), jnp.float32),
    )(x, parts, W1, b1, W2, b2, eps11, gamma, beta)


def kernel(x, edge_index, edge_attr, W1, b1, W2, b2, eps, gamma, beta):
    ei_flat = edge_index.reshape(2 * E)
    parts = _sc_aggr(x, ei_flat, edge_attr)
    eps11 = jnp.reshape(eps, (1, 1)).astype(jnp.float32)
    return _tc_mlp(x, parts, W1,
                   jnp.reshape(b1, (1, D)), W2, jnp.reshape(b2, (1, D)),
                   eps11, jnp.reshape(gamma, (1, D)), jnp.reshape(beta, (1, D)))
